# baseline (speedup 1.0000x reference)
import jax
import jax.numpy as jnp
from jax import lax
from jax.experimental import pallas as pl
from jax.experimental.pallas import tpu as pltpu
from jax.experimental.pallas import tpu_sc as plsc

K = 129
D_MODEL = 1024
NUM_HEADS = 16
SCALE = 32.0

HALF0 = 64
RPS = 8


def _scale_rows(buf, nrows):
    for r in range(nrows):
        def body(i, carry, r=r):
            sl = pl.ds(i * 16, 16)
            buf[r, sl] = buf[r, sl] * SCALE
            return carry
        lax.fori_loop(0, D_MODEL // 16, body, 0)


def _sc_body(wv, ov, shared, buf, sem):
    s = lax.axis_index("s")
    c = lax.axis_index("c")

    @pl.when(s < 8)
    def _stage():
        base = c * HALF0 + s * RPS
        pltpu.sync_copy(wv.at[pl.ds(base, RPS)], buf)
        _scale_rows(buf, RPS)
        pltpu.sync_copy(buf, shared.at[pl.ds(s * RPS, RPS)])

    plsc.subcore_barrier()

    @pl.when(c == 0)
    def _lo():
        pltpu.async_copy(shared.at[pl.ds(0, HALF0)],
                         ov.at[0, s, pl.ds(0, HALF0)], sem).wait()

    @pl.when(c == 1)
    def _hi():
        d = pltpu.async_copy(shared.at[pl.ds(0, HALF0)],
                             ov.at[0, s, pl.ds(HALF0, HALF0)], sem)
        pltpu.sync_copy(wv.at[pl.ds(K - 1, 1)], buf.at[pl.ds(0, 1)])
        _scale_rows(buf, 1)
        pltpu.sync_copy(buf.at[pl.ds(0, 1)], ov.at[0, s, pl.ds(K - 1, 1)])
        d.wait()


def _sc_rel_v(Wv):
    out = jax.ShapeDtypeStruct((1, NUM_HEADS, K, D_MODEL), jnp.float32)
    mesh = plsc.VectorSubcoreMesh(core_axis_name="c", subcore_axis_name="s")
    f = pl.kernel(
        _sc_body,
        out_type=out,
        mesh=mesh,
        scratch_types=[
            pltpu.VMEM_SHARED((HALF0, D_MODEL), jnp.float32),
            pltpu.VMEM((RPS, D_MODEL), jnp.float32),
            pltpu.SemaphoreType.DMA,
        ],
    )
    return f(Wv)


HEADS_PER_STEP = 4


def _tc_body(wq_ref, wk_ref, oq_ref, ok_ref):
    for o_ref, w_ref in ((oq_ref, wq_ref), (ok_ref, wk_ref)):
        w = w_ref[...] * SCALE
        o_ref[0] = jnp.broadcast_to(w[None], (HEADS_PER_STEP, K, D_MODEL))


def _tc_rel_qk(Wq, Wk):
    in_spec = pl.BlockSpec((K, D_MODEL), lambda h: (0, 0))
    out_spec = pl.BlockSpec(
        (1, HEADS_PER_STEP, K, D_MODEL), lambda h: (0, h, 0, 0)
    )
    out_shape = jax.ShapeDtypeStruct((1, NUM_HEADS, K, D_MODEL), jnp.float32)
    return pl.pallas_call(
        _tc_body,
        grid=(NUM_HEADS // HEADS_PER_STEP,),
        in_specs=[in_spec, in_spec],
        out_specs=[out_spec, out_spec],
        out_shape=[out_shape, out_shape],
        compiler_params=pltpu.CompilerParams(
            dimension_semantics=("parallel",)
        ),
    )(Wq, Wk)


def kernel(Wq, Wk, Wv):
    rel_v = _sc_rel_v(Wv)
    rel_q, rel_k = _tc_rel_qk(Wq, Wk)
    return (rel_q, rel_k, rel_v)
